# Initial kernel scaffold; baseline (speedup 1.0000x reference)
import jax
import jax.numpy as jnp
from jax.experimental import pallas as pl


def kernel(graphs, weight):
    raise NotImplementedError("write your pallas kernel here")


# speedup vs baseline: 11.5577x; 11.5577x over previous
import functools

import jax
import jax.numpy as jnp
from jax.experimental import pallas as pl
from jax.experimental.pallas import tpu as pltpu

B, T, C, N = 4, 12, 4, 512
K = 5
ROW_BLK = 256


def _fused_kernel(w_ref, g_ref, o_ref):
    w = w_ref[0, 0, :]
    w = jax.nn.softmax(w)

    g = g_ref[0]
    s = jnp.maximum(jnp.sum(jnp.abs(g), axis=-1, keepdims=True), 1e-12)
    coef = w[:, None, None] / s
    agg = jnp.sum(g * coef, axis=0)

    neg = jnp.float32(-jnp.inf)
    thr = jnp.max(agg, axis=-1, keepdims=True)
    for _ in range(K - 1):
        below = jnp.where(agg < thr, agg, neg)
        thr = jnp.max(below, axis=-1, keepdims=True)

    masked = jnp.where(agg >= thr, agg, 0.0)
    denom = jnp.maximum(jnp.sum(masked, axis=-1, keepdims=True), 1e-12)
    o_ref[0] = masked / denom


@jax.jit
def kernel(graphs, weight):
    g = graphs.reshape(B * T, C, N, N)
    w = jnp.broadcast_to(weight.reshape(1, T, C), (B, T, C)).reshape(B * T, 1, C)

    grid = (B * T, N // ROW_BLK)
    out = pl.pallas_call(
        _fused_kernel,
        grid=grid,
        in_specs=[
            pl.BlockSpec((1, 1, C), lambda bt, ib: (bt, 0, 0)),
            pl.BlockSpec((1, C, ROW_BLK, N), lambda bt, ib: (bt, 0, ib, 0)),
        ],
        out_specs=pl.BlockSpec((1, ROW_BLK, N), lambda bt, ib: (bt, ib, 0)),
        out_shape=jax.ShapeDtypeStruct((B * T, N, N), jnp.float32),
        compiler_params=pltpu.CompilerParams(
            dimension_semantics=("parallel", "parallel"),
        ),
    )(w, g)
    return out.reshape(B, T, N, N)
